# baseline (speedup 1.0000x reference)
import jax
import jax.numpy as jnp
from jax.experimental import pallas as pl

THRESH = 0.1

_TN = (((0,), (0,)), ((), ()))
_NN = (((1,), (0,)), ((), ()))


def _dot(a, b, dims):
    return jax.lax.dot_general(a, b, dims, preferred_element_type=jnp.float32)


def _normalize_cols_bf16(x):
    n = jnp.maximum(jnp.sqrt(jnp.sum(x * x, axis=0, keepdims=True)), 1e-8)
    return (x / n).astype(jnp.bfloat16)


def _hg_kernel(xt_ref, xc1_ref, xc2_ref, w1e_ref, b1e_ref, w2e_ref, b2e_ref,
               yt_ref, yc1_ref, yc2_ref):
    w1e = w1e_ref[...].astype(jnp.bfloat16)
    b1e = b1e_ref[...]
    w2e = w2e_ref[...].astype(jnp.bfloat16)
    b2e = b2e_ref[...]
    nc = w1e.shape[0]

    selfs = (xt_ref[0], xc2_ref[0])
    nbrs = ((xt_ref[1], xc1_ref[0]), (xc1_ref[1], xc2_ref[1]))

    for i in range(2):
        tn = _normalize_cols_bf16(xt_ref[i])
        cab = jnp.concatenate([_normalize_cols_bf16(xc1_ref[i]),
                               _normalize_cols_bf16(xc2_ref[i])], axis=1)

        m = (_dot(tn, cab, _TN) > THRESH).astype(jnp.bfloat16)
        mt = (_dot(cab, tn, _TN) > THRESH).astype(jnp.bfloat16)

        x_self = selfs[i].astype(jnp.bfloat16)
        x_nab = jnp.concatenate([nbrs[i][0].astype(jnp.bfloat16),
                                 nbrs[i][1].astype(jnp.bfloat16)], axis=1)
        t_self = _dot(w1e, x_self, _TN) + b1e
        t_nab = _dot(w1e, x_nab, _TN) + b1e

        s = t_self + _dot(t_nab.astype(jnp.bfloat16), mt, _NN)
        x_edge = (s[:nc] / s[nc:nc + 1]).astype(jnp.bfloat16)
        u = _dot(w2e, x_edge, _TN) + b2e

        stu = _dot(u.astype(jnp.bfloat16), m, _NN)
        s_ab = stu[:nc] / jnp.maximum(stu[nc:nc + 1], 1.0)

        n = tn.shape[1]
        if i == 0:
            yt_ref[0] = u[:nc]
            yt_ref[1] = s_ab[:, :n]
            yc1_ref[0] = s_ab[:, n:]
        else:
            yc2_ref[0] = u[:nc]
            yc1_ref[1] = s_ab[:, :n]
            yc2_ref[1] = s_ab[:, n:]


def kernel(X_target, X_context1, X_context2, W1, b1, W2, b2):
    B, C, Hh, Ww = X_target.shape
    N = Hh * Ww
    xt = X_target.reshape(B, C, N)
    xc1 = X_context1.reshape(B, C, N)
    xc2 = X_context2.reshape(B, C, N)

    zcol = jnp.zeros((C, 1), jnp.float32)
    one = jnp.ones((1, 1), jnp.float32)
    w1e = jnp.concatenate([W1, zcol], axis=1)
    w2e = jnp.concatenate([W2, zcol], axis=1)
    b1e = jnp.concatenate([b1.reshape(C, 1), one], axis=0)
    b2e = jnp.concatenate([b2.reshape(C, 1), one], axis=0)

    shp = jax.ShapeDtypeStruct((B, C, N), jnp.float32)
    yt, yc1, yc2 = pl.pallas_call(
        _hg_kernel,
        out_shape=[shp, shp, shp],
    )(xt, xc1, xc2, w1e, b1e, w2e, b2e)

    rs = lambda a: a.reshape(B, C, Hh, Ww)
    return (rs(yt), rs(yc1), rs(yc2))
